# baseline (speedup 1.0000x reference)
import functools

import jax
import jax.numpy as jnp
from jax import lax
from jax.experimental import pallas as pl
from jax.experimental.pallas import tpu as pltpu
from jax.experimental.pallas import tpu_sc as plsc

_NUM_CLASSES = 100000
_BATCH = 4096
_NC = 2
_NS = 16
_NW = _NC * _NS
_ROWS_PER_W = _BATCH // _NW
_SPAN = _ROWS_PER_W * _NUM_CLASSES
_ZCHUNK = 100000
_NZ = _SPAN // _ZCHUNK
_LANES = 16


def _onehot_sc(idx_hbm, out_hbm, zbuf, idx_v, fidx_v, ones_v, zsem, ssem):
    wid = lax.axis_index("s") * _NC + lax.axis_index("c")
    base_row = wid * _ROWS_PER_W
    base_elem = wid * _SPAN

    zeros16 = jnp.zeros((_LANES,), jnp.float32)

    def _zb(i, _):
        zbuf[pl.ds(i * _LANES, _LANES)] = zeros16
        return 0

    lax.fori_loop(0, _ZCHUNK // _LANES, _zb, 0)

    pltpu.sync_copy(idx_hbm.at[pl.ds(base_row, _ROWS_PER_W)], idx_v)
    iota = lax.iota(jnp.int32, _LANES)
    ones16 = jnp.ones((_LANES,), jnp.float32)
    for g in range(_ROWS_PER_W // _LANES):
        rows = base_row + g * _LANES + iota
        hot = idx_v[pl.ds(g * _LANES, _LANES)]
        fidx_v[pl.ds(g * _LANES, _LANES)] = rows * _NUM_CLASSES + hot
        ones_v[pl.ds(g * _LANES, _LANES)] = ones16

    def _zcopy(i):
        return pltpu.make_async_copy(
            zbuf, out_hbm.at[pl.ds(base_elem + i * _ZCHUNK, _ZCHUNK)], zsem
        )

    def _fire(i, _):
        _zcopy(i).start()
        return 0

    lax.fori_loop(0, _NZ, _fire, 0)

    def _drain(i, _):
        _zcopy(i).wait()
        return 0

    lax.fori_loop(0, _NZ, _drain, 0)

    pltpu.make_async_copy(ones_v, out_hbm.at[fidx_v], ssem).start()
    pltpu.make_async_copy(ones_v, out_hbm.at[fidx_v], ssem).wait()


def kernel(idx):
    idx2 = idx.astype(jnp.int32)
    mesh = plsc.VectorSubcoreMesh(core_axis_name="c", subcore_axis_name="s")
    run = pl.kernel(
        _onehot_sc,
        out_type=jax.ShapeDtypeStruct((_BATCH * _NUM_CLASSES,), jnp.float32),
        mesh=mesh,
        scratch_types=[
            pltpu.VMEM((_ZCHUNK,), jnp.float32),
            pltpu.VMEM((_ROWS_PER_W,), jnp.int32),
            pltpu.VMEM((_ROWS_PER_W,), jnp.int32),
            pltpu.VMEM((_ROWS_PER_W,), jnp.float32),
            pltpu.SemaphoreType.DMA,
            pltpu.SemaphoreType.DMA,
        ],
    )
    flat = run(idx2)
    return flat.reshape(_BATCH, _NUM_CLASSES)


# speedup vs baseline: 1.7711x; 1.7711x over previous
import jax
import jax.numpy as jnp
from jax import lax
from jax.experimental import pallas as pl
from jax.experimental.pallas import tpu as pltpu
from jax.experimental.pallas import tpu_sc as plsc

_NUM_CLASSES = 100000
_BATCH = 4096
_NC = 2
_NS = 16
_NW = _NC * _NS
_ROWS_PER_W = _BATCH // _NW
_LANES = 16
_NSLOT = 8


def _zero_sc(out_hbm, zbuf, sem):
    wid = lax.axis_index("s") * _NC + lax.axis_index("c")
    base_row = wid * _ROWS_PER_W

    zeros16 = jnp.zeros((_LANES,), jnp.float32)

    def _zb(i, _):
        zbuf[pl.ds(i * _LANES, _LANES)] = zeros16
        return 0

    lax.fori_loop(0, _NUM_CLASSES // _LANES, _zb, 0)

    def _zcopy(r):
        return pltpu.make_async_copy(zbuf, out_hbm.at[base_row + r], sem)

    def _fire(r, _):
        _zcopy(r).start()
        return 0

    lax.fori_loop(0, _ROWS_PER_W, _fire, 0)

    def _drain(r, _):
        _zcopy(r).wait()
        return 0

    lax.fori_loop(0, _ROWS_PER_W, _drain, 0)


def _ones_tc(idx_ref, zeroed_ref, out_ref, stage_ref, sems):
    iota = lax.broadcasted_iota(jnp.int32, (1, 128), 1)

    def _copy(r, slot):
        c = idx_ref[r]
        cb = (c // 128) * 128
        return pltpu.make_async_copy(
            stage_ref.at[pl.ds(slot, 1), :],
            out_ref.at[pl.ds(r, 1), pl.ds(cb, 128)],
            sems.at[slot],
        )

    def _row(r, _):
        slot = lax.rem(r, _NSLOT)

        @pl.when(r >= _NSLOT)
        def _():
            _copy(r - _NSLOT, slot).wait()

        lane = lax.rem(idx_ref[r], 128)
        stage_ref[pl.ds(slot, 1), :] = (iota == lane).astype(jnp.float32)
        _copy(r, slot).start()
        return 0

    lax.fori_loop(0, _BATCH, _row, 0)

    def _draintail(k, _):
        r = _BATCH - _NSLOT + k
        _copy(r, lax.rem(r, _NSLOT)).wait()
        return 0

    lax.fori_loop(0, _NSLOT, _draintail, 0)


def kernel(idx):
    idx2 = idx.astype(jnp.int32)
    mesh = plsc.VectorSubcoreMesh(core_axis_name="c", subcore_axis_name="s")
    zeroed = pl.kernel(
        _zero_sc,
        out_type=jax.ShapeDtypeStruct((_BATCH, _NUM_CLASSES), jnp.float32),
        mesh=mesh,
        scratch_types=[
            pltpu.VMEM((_NUM_CLASSES,), jnp.float32),
            pltpu.SemaphoreType.DMA,
        ],
    )()

    return pl.pallas_call(
        _ones_tc,
        in_specs=[
            pl.BlockSpec(memory_space=pltpu.SMEM),
            pl.BlockSpec(memory_space=pl.ANY),
        ],
        out_specs=pl.BlockSpec(memory_space=pl.ANY),
        out_shape=jax.ShapeDtypeStruct((_BATCH, _NUM_CLASSES), jnp.float32),
        scratch_shapes=[
            pltpu.VMEM((_NSLOT, 128), jnp.float32),
            pltpu.SemaphoreType.DMA((_NSLOT,)),
        ],
        input_output_aliases={1: 0},
    )(idx2, zeroed)


# speedup vs baseline: 1.9409x; 1.0959x over previous
import jax
import jax.numpy as jnp
from jax import lax
from jax.experimental import pallas as pl
from jax.experimental.pallas import tpu as pltpu
from jax.experimental.pallas import tpu_sc as plsc

_NUM_CLASSES = 100000
_BATCH = 4096
_NC = 2
_NS = 16
_NW = _NC * _NS
_ROWS_PER_W = _BATCH // _NW
_LANES = 16
_NSLOT = 64


def _zero_sc(out_hbm, zbuf, sem):
    wid = lax.axis_index("s") * _NC + lax.axis_index("c")
    base_row = wid * _ROWS_PER_W

    zeros16 = jnp.zeros((_LANES,), jnp.float32)

    def _zb(i, _):
        zbuf[pl.ds(i * _LANES, _LANES)] = zeros16
        return 0

    lax.fori_loop(0, _NUM_CLASSES // _LANES, _zb, 0)

    def _zcopy(r):
        return pltpu.make_async_copy(zbuf, out_hbm.at[base_row + r], sem)

    def _fire(r, _):
        _zcopy(r).start()
        return 0

    lax.fori_loop(0, _ROWS_PER_W, _fire, 0)

    def _drain(r, _):
        _zcopy(r).wait()
        return 0

    lax.fori_loop(0, _ROWS_PER_W, _drain, 0)


def _ones_tc(idx_ref, zeroed_ref, out_ref, stage_ref, sems):
    iota = lax.broadcasted_iota(jnp.int32, (1, 128), 1)

    def _copy(r, slot):
        c = idx_ref[r]
        cb = (c // 128) * 128
        return pltpu.make_async_copy(
            stage_ref.at[pl.ds(slot, 1), :],
            out_ref.at[pl.ds(r, 1), pl.ds(cb, 128)],
            sems.at[slot],
        )

    def _row(r, _):
        slot = lax.rem(r, _NSLOT)

        @pl.when(r >= _NSLOT)
        def _():
            _copy(r - _NSLOT, slot).wait()

        lane = lax.rem(idx_ref[r], 128)
        stage_ref[pl.ds(slot, 1), :] = (iota == lane).astype(jnp.float32)
        _copy(r, slot).start()
        return 0

    lax.fori_loop(0, _BATCH, _row, 0)

    def _draintail(k, _):
        r = _BATCH - _NSLOT + k
        _copy(r, lax.rem(r, _NSLOT)).wait()
        return 0

    lax.fori_loop(0, _NSLOT, _draintail, 0)


def kernel(idx):
    idx2 = idx.astype(jnp.int32)
    mesh = plsc.VectorSubcoreMesh(core_axis_name="c", subcore_axis_name="s")
    zeroed = pl.kernel(
        _zero_sc,
        out_type=jax.ShapeDtypeStruct((_BATCH, _NUM_CLASSES), jnp.float32),
        mesh=mesh,
        scratch_types=[
            pltpu.VMEM((_NUM_CLASSES,), jnp.float32),
            pltpu.SemaphoreType.DMA,
        ],
    )()

    return pl.pallas_call(
        _ones_tc,
        in_specs=[
            pl.BlockSpec(memory_space=pltpu.SMEM),
            pl.BlockSpec(memory_space=pl.ANY),
        ],
        out_specs=pl.BlockSpec(memory_space=pl.ANY),
        out_shape=jax.ShapeDtypeStruct((_BATCH, _NUM_CLASSES), jnp.float32),
        scratch_shapes=[
            pltpu.VMEM((_NSLOT, 128), jnp.float32),
            pltpu.SemaphoreType.DMA((_NSLOT,)),
        ],
        input_output_aliases={1: 0},
    )(idx2, zeroed)


# speedup vs baseline: 1.9563x; 1.0079x over previous
import jax
import jax.numpy as jnp
from jax import lax
from jax.experimental import pallas as pl
from jax.experimental.pallas import tpu as pltpu
from jax.experimental.pallas import tpu_sc as plsc

_NUM_CLASSES = 100000
_BATCH = 4096
_NC = 2
_NS = 16
_NW = _NC * _NS
_ROWS_PER_W = _BATCH // _NW
_LANES = 16
_NSLOT = 128


def _zero_sc(out_hbm, zbuf, sem):
    wid = lax.axis_index("s") * _NC + lax.axis_index("c")
    base_row = wid * _ROWS_PER_W

    zeros16 = jnp.zeros((_LANES,), jnp.float32)

    def _zb(i, _):
        zbuf[pl.ds(i * _LANES, _LANES)] = zeros16
        return 0

    lax.fori_loop(0, _NUM_CLASSES // _LANES, _zb, 0)

    def _zcopy(r):
        return pltpu.make_async_copy(zbuf, out_hbm.at[base_row + r], sem)

    def _fire(r, _):
        _zcopy(r).start()
        return 0

    lax.fori_loop(0, _ROWS_PER_W, _fire, 0)

    def _drain(r, _):
        _zcopy(r).wait()
        return 0

    lax.fori_loop(0, _ROWS_PER_W, _drain, 0)


def _ones_tc(idx_ref, zeroed_ref, out_ref, stage_ref, sems):
    iota = lax.broadcasted_iota(jnp.int32, (1, 128), 1)

    def _start(r, slot):
        c = idx_ref[r]
        cb = (c // 128) * 128
        pltpu.make_async_copy(
            stage_ref.at[pl.ds(slot, 1), :],
            out_ref.at[pl.ds(r, 1), pl.ds(cb, 128)],
            sems.at[slot],
        ).start()

    def _wait(slot):
        pltpu.make_async_copy(
            stage_ref.at[pl.ds(slot, 1), :],
            out_ref.at[pl.ds(0, 1), pl.ds(0, 128)],
            sems.at[slot],
        ).wait()

    def _row(r, _):
        slot = lax.rem(r, _NSLOT)

        @pl.when(r >= _NSLOT)
        def _():
            _wait(slot)

        lane = lax.rem(idx_ref[r], 128)
        stage_ref[pl.ds(slot, 1), :] = (iota == lane).astype(jnp.float32)
        _start(r, slot)
        return 0

    lax.fori_loop(0, _BATCH, _row, 0, unroll=8)

    def _draintail(k, _):
        _wait(lax.rem(_BATCH - _NSLOT + k, _NSLOT))
        return 0

    lax.fori_loop(0, _NSLOT, _draintail, 0, unroll=8)


def kernel(idx):
    idx2 = idx.astype(jnp.int32)
    mesh = plsc.VectorSubcoreMesh(core_axis_name="c", subcore_axis_name="s")
    zeroed = pl.kernel(
        _zero_sc,
        out_type=jax.ShapeDtypeStruct((_BATCH, _NUM_CLASSES), jnp.float32),
        mesh=mesh,
        scratch_types=[
            pltpu.VMEM((_NUM_CLASSES,), jnp.float32),
            pltpu.SemaphoreType.DMA,
        ],
    )()

    return pl.pallas_call(
        _ones_tc,
        in_specs=[
            pl.BlockSpec(memory_space=pltpu.SMEM),
            pl.BlockSpec(memory_space=pl.ANY),
        ],
        out_specs=pl.BlockSpec(memory_space=pl.ANY),
        out_shape=jax.ShapeDtypeStruct((_BATCH, _NUM_CLASSES), jnp.float32),
        scratch_shapes=[
            pltpu.VMEM((_NSLOT, 128), jnp.float32),
            pltpu.SemaphoreType.DMA((_NSLOT,)),
        ],
        input_output_aliases={1: 0},
    )(idx2, zeroed)


# speedup vs baseline: 2.1095x; 1.0783x over previous
import jax
import jax.numpy as jnp
from jax.experimental import pallas as pl

_NUM_CLASSES = 100000
_BLOCK_COLS = 1024


def _onehot_block(idx_ref, out_ref):
    j = pl.program_id(0)
    base = j * _BLOCK_COLS
    idx = idx_ref[:]
    b = idx.shape[0]
    cols = jax.lax.broadcasted_iota(jnp.int32, (b, _BLOCK_COLS), 1) + base
    out_ref[:, :] = (idx == cols).astype(jnp.float32)


def kernel(idx):
    b = idx.shape[0]
    idx2 = idx.astype(jnp.int32).reshape(b, 1)
    grid = (pl.cdiv(_NUM_CLASSES, _BLOCK_COLS),)
    return pl.pallas_call(
        _onehot_block,
        grid=grid,
        in_specs=[pl.BlockSpec((b, 1), lambda j: (0, 0))],
        out_specs=pl.BlockSpec((b, _BLOCK_COLS), lambda j: (0, j)),
        out_shape=jax.ShapeDtypeStruct((b, _NUM_CLASSES), jnp.float32),
    )(idx2)
